# initial kernel scaffold (unmeasured)
import jax
import jax.numpy as jnp
from jax import lax
from jax.experimental import pallas as pl
from jax.experimental.pallas import tpu as pltpu

N_DEV = 4


def kernel(x, w_mat):
    m_per, k = x.shape
    _, n_per = w_mat.shape
    M = N_DEV * m_per

    def body(x_ref, w_ref, out_ref, xg_ref, amax_ref,
             send_sems, recv_sems, a_send_sems, a_recv_sems):
        my = lax.axis_index("i")
        left = (my - 1) % N_DEV
        right = (my + 1) % N_DEV

        barrier_sem = pltpu.get_barrier_semaphore()
        for nbr in (left, right):
            pl.semaphore_signal(
                barrier_sem, inc=1,
                device_id=(nbr,), device_id_type=pl.DeviceIdType.MESH,
            )
        pl.semaphore_wait(barrier_sem, 2)

        xg_ref[pl.ds(my * m_per, m_per), :] = x_ref[...].astype(jnp.bfloat16)

        for h in range(N_DEV - 1):
            c = (my - h) % N_DEV
            sl = pl.ds(c * m_per, m_per)
            rdma = pltpu.make_async_remote_copy(
                src_ref=xg_ref.at[sl, :],
                dst_ref=xg_ref.at[sl, :],
                send_sem=send_sems.at[h],
                recv_sem=recv_sems.at[h],
                device_id=(right,),
                device_id_type=pl.DeviceIdType.MESH,
            )
            rdma.start()
            rdma.wait()

        w_bf = w_ref[...].astype(jnp.bfloat16)
        y = jnp.dot(xg_ref[...], w_bf, preferred_element_type=jnp.float32)
        y = jnp.maximum(y, 0.0)

        local_amax = jnp.max(y)
        amax_ref[pl.ds(my, 1), :] = jnp.full((1, 128), local_amax, jnp.float32)
        amax_rdmas = []
        for off in range(1, N_DEV):
            p = (my + off) % N_DEV
            r = pltpu.make_async_remote_copy(
                src_ref=amax_ref.at[pl.ds(my, 1), :],
                dst_ref=amax_ref.at[pl.ds(my, 1), :],
                send_sem=a_send_sems.at[off - 1],
                recv_sem=a_recv_sems.at[off - 1],
                device_id=(p,),
                device_id_type=pl.DeviceIdType.MESH,
            )
            r.start()
            amax_rdmas.append(r)
        for r in amax_rdmas:
            r.wait()

        gmax = jnp.max(amax_ref[...])
        scale = gmax / 448.0
        q = (y / scale).astype(jnp.float8_e4m3fn).astype(jnp.float32)
        out_ref[...] = q * scale

    return pl.pallas_call(
        body,
        out_shape=jax.ShapeDtypeStruct((M, n_per), jnp.float32),
        in_specs=[
            pl.BlockSpec(memory_space=pltpu.VMEM),
            pl.BlockSpec(memory_space=pltpu.VMEM),
        ],
        out_specs=pl.BlockSpec(memory_space=pltpu.VMEM),
        scratch_shapes=[
            pltpu.VMEM((M, k), jnp.bfloat16),
            pltpu.VMEM((N_DEV, 128), jnp.float32),
            pltpu.SemaphoreType.DMA((N_DEV - 1,)),
            pltpu.SemaphoreType.DMA((N_DEV - 1,)),
            pltpu.SemaphoreType.DMA((N_DEV - 1,)),
            pltpu.SemaphoreType.DMA((N_DEV - 1,)),
        ],
        compiler_params=pltpu.CompilerParams(collective_id=0),
    )(x, w_mat)


# baseline (device time: 329697 ns/iter reference)
import jax
import jax.numpy as jnp
from jax import lax
from jax.experimental import pallas as pl
from jax.experimental.pallas import tpu as pltpu

N_DEV = 4


def kernel(x, w_mat):
    m_per, k = x.shape
    _, n_per = w_mat.shape
    M = N_DEV * m_per

    x = x.astype(jnp.bfloat16)
    w_mat = w_mat.astype(jnp.bfloat16)

    def body(x_ref, w_ref, out_ref, xg_ref, amax_ref,
             send_sems, recv_sems, a_send_sems, a_recv_sems):
        my = lax.axis_index("i")
        left = (my - 1) % N_DEV
        right = (my + 1) % N_DEV

        barrier_sem = pltpu.get_barrier_semaphore()
        for nbr in (left, right):
            pl.semaphore_signal(
                barrier_sem, inc=1,
                device_id=(nbr,), device_id_type=pl.DeviceIdType.MESH,
            )
        pl.semaphore_wait(barrier_sem, 2)

        xg_ref[pl.ds(my * m_per, m_per), :] = x_ref[...]

        for h in range(N_DEV - 1):
            c = (my - h) % N_DEV
            sl = pl.ds(c * m_per, m_per)
            rdma = pltpu.make_async_remote_copy(
                src_ref=xg_ref.at[sl, :],
                dst_ref=xg_ref.at[sl, :],
                send_sem=send_sems.at[h],
                recv_sem=recv_sems.at[h],
                device_id=(right,),
                device_id_type=pl.DeviceIdType.MESH,
            )
            rdma.start()
            rdma.wait()

        local_amax = jnp.float32(0.0)
        for j in range(N_DEV):
            yj = jnp.dot(
                xg_ref[j * m_per:(j + 1) * m_per, :], w_ref[...],
                preferred_element_type=jnp.float32,
            )
            yj = jnp.maximum(yj, 0.0)
            out_ref[j * m_per:(j + 1) * m_per, :] = yj
            local_amax = jnp.maximum(local_amax, jnp.max(yj))

        amax_ref[pl.ds(my, 1), :] = jnp.full((1, 128), local_amax, jnp.float32)
        amax_rdmas = []
        for off in range(1, N_DEV):
            p = (my + off) % N_DEV
            r = pltpu.make_async_remote_copy(
                src_ref=amax_ref.at[pl.ds(my, 1), :],
                dst_ref=amax_ref.at[pl.ds(my, 1), :],
                send_sem=a_send_sems.at[off - 1],
                recv_sem=a_recv_sems.at[off - 1],
                device_id=(p,),
                device_id_type=pl.DeviceIdType.MESH,
            )
            r.start()
            amax_rdmas.append(r)
        for r in amax_rdmas:
            r.wait()

        gmax = jnp.max(amax_ref[...])
        scale = gmax / 448.0
        inv_scale = 448.0 / gmax
        for j in range(N_DEV):
            blk = out_ref[j * m_per:(j + 1) * m_per, :]
            q = (blk * inv_scale).astype(jnp.float8_e4m3fn).astype(jnp.float32)
            out_ref[j * m_per:(j + 1) * m_per, :] = q * scale

    return pl.pallas_call(
        body,
        out_shape=jax.ShapeDtypeStruct((M, n_per), jnp.float32),
        in_specs=[
            pl.BlockSpec(memory_space=pltpu.VMEM),
            pl.BlockSpec(memory_space=pltpu.VMEM),
        ],
        out_specs=pl.BlockSpec(memory_space=pltpu.VMEM),
        scratch_shapes=[
            pltpu.VMEM((M, k), jnp.bfloat16),
            pltpu.VMEM((N_DEV, 128), jnp.float32),
            pltpu.SemaphoreType.DMA((N_DEV - 1,)),
            pltpu.SemaphoreType.DMA((N_DEV - 1,)),
            pltpu.SemaphoreType.DMA((N_DEV - 1,)),
            pltpu.SemaphoreType.DMA((N_DEV - 1,)),
        ],
        compiler_params=pltpu.CompilerParams(
            collective_id=0,
            vmem_limit_bytes=64 * 1024 * 1024,
        ),
    )(x, w_mat)


# device time: 180190 ns/iter; 1.8297x vs baseline; 1.8297x over previous
import jax
import jax.numpy as jnp
from jax import lax
from jax.experimental import pallas as pl
from jax.experimental.pallas import tpu as pltpu

N_DEV = 4


def kernel(x, w_mat):
    m_per, k = x.shape
    _, n_per = w_mat.shape
    M = N_DEV * m_per

    x = x.astype(jnp.bfloat16)
    w_mat = w_mat.astype(jnp.bfloat16)

    def body(x_ref, w_ref, out_ref, xg_ref, amax_ref,
             send_sems, recv_sems, a_send_sems, a_recv_sems):
        my = lax.axis_index("i")
        left = (my - 1) % N_DEV
        right = (my + 1) % N_DEV

        barrier_sem = pltpu.get_barrier_semaphore()
        for nbr in (left, right):
            pl.semaphore_signal(
                barrier_sem, inc=1,
                device_id=(nbr,), device_id_type=pl.DeviceIdType.MESH,
            )
        pl.semaphore_wait(barrier_sem, 2)

        xg_ref[pl.ds(my * m_per, m_per), :] = x_ref[...]

        m_half = m_per // 2

        def mk(sl, sem_idx, target):
            return pltpu.make_async_remote_copy(
                src_ref=xg_ref.at[sl, :],
                dst_ref=xg_ref.at[sl, :],
                send_sem=send_sems.at[sem_idx],
                recv_sem=recv_sems.at[sem_idx],
                device_id=(target,),
                device_id_type=pl.DeviceIdType.MESH,
            )

        def gemm(c):
            sl = pl.ds(c * m_per, m_per)
            yj = jnp.dot(
                xg_ref[sl, :], w_ref[...],
                preferred_element_type=jnp.float32,
            )
            yj = jnp.maximum(yj, 0.0)
            out_ref[sl, :] = yj
            return jnp.max(yj)

        own = pl.ds(my * m_per, m_per)
        snd0r = mk(own, 0, right)
        snd0l = mk(own, 1, left)
        snd0r.start()
        snd0l.start()

        local_amax = gemm(my)

        snd0r.wait()
        snd0l.wait()

        cl = (my - 1) % N_DEV
        cr = (my + 1) % N_DEV
        snd1r = mk(pl.ds(cl * m_per, m_half), 2, right)
        snd1l = mk(pl.ds(cr * m_per + m_half, m_half), 3, left)
        snd1r.start()
        snd1l.start()

        local_amax = jnp.maximum(local_amax, gemm(cl))
        local_amax = jnp.maximum(local_amax, gemm(cr))

        snd1r.wait()
        snd1l.wait()

        local_amax = jnp.maximum(local_amax, gemm((my + 2) % N_DEV))

        amax_ref[pl.ds(my, 1), :] = jnp.full((1, 128), local_amax, jnp.float32)
        amax_rdmas = []
        for off in range(1, N_DEV):
            p = (my + off) % N_DEV
            r = pltpu.make_async_remote_copy(
                src_ref=amax_ref.at[pl.ds(my, 1), :],
                dst_ref=amax_ref.at[pl.ds(my, 1), :],
                send_sem=a_send_sems.at[off - 1],
                recv_sem=a_recv_sems.at[off - 1],
                device_id=(p,),
                device_id_type=pl.DeviceIdType.MESH,
            )
            r.start()
            amax_rdmas.append(r)
        for r in amax_rdmas:
            r.wait()

        gmax = jnp.max(amax_ref[...])
        scale = gmax / 448.0
        inv_scale = 448.0 / gmax
        for j in range(N_DEV):
            blk = out_ref[j * m_per:(j + 1) * m_per, :]
            q = (blk * inv_scale).astype(jnp.float8_e4m3fn).astype(jnp.float32)
            out_ref[j * m_per:(j + 1) * m_per, :] = q * scale

    return pl.pallas_call(
        body,
        out_shape=jax.ShapeDtypeStruct((M, n_per), jnp.float32),
        in_specs=[
            pl.BlockSpec(memory_space=pltpu.VMEM),
            pl.BlockSpec(memory_space=pltpu.VMEM),
        ],
        out_specs=pl.BlockSpec(memory_space=pltpu.VMEM),
        scratch_shapes=[
            pltpu.VMEM((M, k), jnp.bfloat16),
            pltpu.VMEM((N_DEV, 128), jnp.float32),
            pltpu.SemaphoreType.DMA((4,)),
            pltpu.SemaphoreType.DMA((4,)),
            pltpu.SemaphoreType.DMA((N_DEV - 1,)),
            pltpu.SemaphoreType.DMA((N_DEV - 1,)),
        ],
        compiler_params=pltpu.CompilerParams(
            collective_id=0,
            vmem_limit_bytes=64 * 1024 * 1024,
        ),
    )(x, w_mat)


# device time: 165367 ns/iter; 1.9937x vs baseline; 1.0896x over previous
import jax
import jax.numpy as jnp
from jax import lax
from jax.experimental import pallas as pl
from jax.experimental.pallas import tpu as pltpu

N_DEV = 4
XPIECES = 4
WPIECES = 4


def kernel(x, w_mat):
    m_per, k = x.shape
    _, n_per = w_mat.shape
    M = N_DEV * m_per
    xp = m_per // XPIECES
    wp = k // WPIECES

    def body(x_hbm, w_hbm, out_ref, xg_ref, wb_ref, xstage, wstage, amax_ref,
             xsems, wsems, send_sems, recv_sems, a_send_sems, a_recv_sems):
        my = lax.axis_index("i")
        left = (my - 1) % N_DEV
        right = (my + 1) % N_DEV

        barrier_sem = pltpu.get_barrier_semaphore()
        for nbr in (left, right):
            pl.semaphore_signal(
                barrier_sem, inc=1,
                device_id=(nbr,), device_id_type=pl.DeviceIdType.MESH,
            )

        xdmas = [
            pltpu.make_async_copy(
                x_hbm.at[pl.ds(t * xp, xp), :], xstage.at[t % 2], xsems.at[t % 2]
            )
            for t in range(XPIECES)
        ]
        xdmas[0].start()
        for t in range(XPIECES):
            if t + 1 < XPIECES:
                xdmas[t + 1].start()
            xdmas[t].wait()
            xg_ref[pl.ds(my * m_per + t * xp, xp), :] = (
                xstage[t % 2].astype(jnp.bfloat16)
            )

        pl.semaphore_wait(barrier_sem, 2)

        m_half = m_per // 2

        def mk(sl, sem_idx, target):
            return pltpu.make_async_remote_copy(
                src_ref=xg_ref.at[sl, :],
                dst_ref=xg_ref.at[sl, :],
                send_sem=send_sems.at[sem_idx],
                recv_sem=recv_sems.at[sem_idx],
                device_id=(target,),
                device_id_type=pl.DeviceIdType.MESH,
            )

        def gemm(c):
            sl = pl.ds(c * m_per, m_per)
            yj = jnp.dot(
                xg_ref[sl, :], wb_ref[...],
                preferred_element_type=jnp.float32,
            )
            yj = jnp.maximum(yj, 0.0)
            out_ref[sl, :] = yj
            return jnp.max(yj)

        own = pl.ds(my * m_per, m_per)
        snd0r = mk(own, 0, right)
        snd0l = mk(own, 1, left)
        snd0r.start()
        snd0l.start()

        wdmas = [
            pltpu.make_async_copy(
                w_hbm.at[pl.ds(t * wp, wp), :], wstage.at[t % 2], wsems.at[t % 2]
            )
            for t in range(WPIECES)
        ]
        wdmas[0].start()
        for t in range(WPIECES):
            if t + 1 < WPIECES:
                wdmas[t + 1].start()
            wdmas[t].wait()
            wb_ref[pl.ds(t * wp, wp), :] = wstage[t % 2].astype(jnp.bfloat16)

        local_amax = gemm(my)

        snd0r.wait()
        snd0l.wait()

        cl = (my - 1) % N_DEV
        cr = (my + 1) % N_DEV
        snd1r = mk(pl.ds(cl * m_per, m_half), 2, right)
        snd1l = mk(pl.ds(cr * m_per + m_half, m_half), 3, left)
        snd1r.start()
        snd1l.start()

        local_amax = jnp.maximum(local_amax, gemm(cl))
        local_amax = jnp.maximum(local_amax, gemm(cr))

        snd1r.wait()
        snd1l.wait()

        local_amax = jnp.maximum(local_amax, gemm((my + 2) % N_DEV))

        amax_ref[pl.ds(my, 1), :] = jnp.full((1, 128), local_amax, jnp.float32)
        amax_rdmas = []
        for off in range(1, N_DEV):
            p = (my + off) % N_DEV
            r = pltpu.make_async_remote_copy(
                src_ref=amax_ref.at[pl.ds(my, 1), :],
                dst_ref=amax_ref.at[pl.ds(my, 1), :],
                send_sem=a_send_sems.at[off - 1],
                recv_sem=a_recv_sems.at[off - 1],
                device_id=(p,),
                device_id_type=pl.DeviceIdType.MESH,
            )
            r.start()
            amax_rdmas.append(r)
        for r in amax_rdmas:
            r.wait()

        gmax = jnp.max(amax_ref[...])
        scale = gmax / 448.0
        inv_scale = 448.0 / gmax
        for j in range(N_DEV):
            blk = out_ref[j * m_per:(j + 1) * m_per, :]
            q = (blk * inv_scale).astype(jnp.float8_e4m3fn).astype(jnp.float32)
            out_ref[j * m_per:(j + 1) * m_per, :] = q * scale

    return pl.pallas_call(
        body,
        out_shape=jax.ShapeDtypeStruct((M, n_per), jnp.float32),
        in_specs=[
            pl.BlockSpec(memory_space=pl.ANY),
            pl.BlockSpec(memory_space=pl.ANY),
        ],
        out_specs=pl.BlockSpec(memory_space=pltpu.VMEM),
        scratch_shapes=[
            pltpu.VMEM((M, k), jnp.bfloat16),
            pltpu.VMEM((k, n_per), jnp.bfloat16),
            pltpu.VMEM((2, m_per // XPIECES, k), jnp.float32),
            pltpu.VMEM((2, k // WPIECES, n_per), jnp.float32),
            pltpu.VMEM((N_DEV, 128), jnp.float32),
            pltpu.SemaphoreType.DMA((2,)),
            pltpu.SemaphoreType.DMA((2,)),
            pltpu.SemaphoreType.DMA((4,)),
            pltpu.SemaphoreType.DMA((4,)),
            pltpu.SemaphoreType.DMA((N_DEV - 1,)),
            pltpu.SemaphoreType.DMA((N_DEV - 1,)),
        ],
        compiler_params=pltpu.CompilerParams(
            collective_id=0,
            vmem_limit_bytes=64 * 1024 * 1024,
        ),
    )(x, w_mat)


# device time: 158353 ns/iter; 2.0820x vs baseline; 1.0443x over previous
import jax
import jax.numpy as jnp
from jax import lax
from jax.experimental import pallas as pl
from jax.experimental.pallas import tpu as pltpu

N_DEV = 4
XPIECES = 4
WPIECES = 4


def kernel(x, w_mat):
    m_per, k = x.shape
    _, n_per = w_mat.shape
    M = N_DEV * m_per
    xp = m_per // XPIECES
    wp = k // WPIECES

    def body(x_hbm, w_hbm, out_ref, xg_ref, wb_ref, xstage, wstage, amax_ref,
             xsems, wsems, send_sems, recv_sems, a_send_sems, a_recv_sems):
        my = lax.axis_index("i")
        left = (my - 1) % N_DEV
        right = (my + 1) % N_DEV

        barrier_sem = pltpu.get_barrier_semaphore()
        for nbr in (left, right):
            pl.semaphore_signal(
                barrier_sem, inc=1,
                device_id=(nbr,), device_id_type=pl.DeviceIdType.MESH,
            )

        def mk(sl, sem_idx, target):
            return pltpu.make_async_remote_copy(
                src_ref=xg_ref.at[sl, :],
                dst_ref=xg_ref.at[sl, :],
                send_sem=send_sems.at[sem_idx],
                recv_sem=recv_sems.at[sem_idx],
                device_id=(target,),
                device_id_type=pl.DeviceIdType.MESH,
            )

        def gemm(row_start, rows):
            sl = pl.ds(row_start, rows)
            yj = jnp.dot(
                xg_ref[sl, :], wb_ref[...],
                preferred_element_type=jnp.float32,
            )
            yj = jnp.maximum(yj, 0.0)
            out_ref[sl, :] = yj
            return jnp.max(yj)

        xdmas = [
            pltpu.make_async_copy(
                x_hbm.at[pl.ds(t * xp, xp), :], xstage.at[t % 2], xsems.at[t % 2]
            )
            for t in range(XPIECES)
        ]
        xdmas[0].start()
        hop0 = []
        for t in range(XPIECES):
            if t + 1 < XPIECES:
                xdmas[t + 1].start()
            xdmas[t].wait()
            xg_ref[pl.ds(my * m_per + t * xp, xp), :] = (
                xstage[t % 2].astype(jnp.bfloat16)
            )
            if t == 0:
                pl.semaphore_wait(barrier_sem, 2)
            psl = pl.ds(my * m_per + t * xp, xp)
            sr = mk(psl, t, right)
            sl_ = mk(psl, 4 + t, left)
            sr.start()
            sl_.start()
            hop0.append((sr, sl_))

        wdmas = [
            pltpu.make_async_copy(
                w_hbm.at[pl.ds(t * wp, wp), :], wstage.at[t % 2], wsems.at[t % 2]
            )
            for t in range(WPIECES)
        ]
        wdmas[0].start()
        for t in range(WPIECES):
            if t + 1 < WPIECES:
                wdmas[t + 1].start()
            wdmas[t].wait()
            wb_ref[pl.ds(t * wp, wp), :] = wstage[t % 2].astype(jnp.bfloat16)

        local_amax = gemm(my * m_per, m_per)

        cl = (my - 1) % N_DEV
        cr = (my + 1) % N_DEV
        hop1 = []
        for u in range(2):
            hop0[u][0].wait()
            f = mk(pl.ds(cl * m_per + u * xp, xp), 8 + u, right)
            f.start()
            hop1.append(f)
        hop0[2][0].wait()
        hop0[3][0].wait()
        hop0[0][1].wait()
        hop0[1][1].wait()
        for u in range(2):
            hop0[2 + u][1].wait()
            f = mk(pl.ds(cr * m_per + (2 + u) * xp, xp), 10 + u, left)
            f.start()
            hop1.append(f)

        local_amax = jnp.maximum(local_amax, gemm(cl * m_per, m_per))
        local_amax = jnp.maximum(local_amax, gemm(cr * m_per, m_per))

        d = (my + 2) % N_DEV
        for u in range(2):
            hop1[u].wait()
            local_amax = jnp.maximum(
                local_amax, gemm(d * m_per + u * xp, xp))
            hop1[2 + u].wait()
            local_amax = jnp.maximum(
                local_amax, gemm(d * m_per + (2 + u) * xp, xp))

        amax_ref[pl.ds(my, 1), :] = jnp.full((1, 128), local_amax, jnp.float32)
        amax_rdmas = []
        for off in range(1, N_DEV):
            p = (my + off) % N_DEV
            r = pltpu.make_async_remote_copy(
                src_ref=amax_ref.at[pl.ds(my, 1), :],
                dst_ref=amax_ref.at[pl.ds(my, 1), :],
                send_sem=a_send_sems.at[off - 1],
                recv_sem=a_recv_sems.at[off - 1],
                device_id=(p,),
                device_id_type=pl.DeviceIdType.MESH,
            )
            r.start()
            amax_rdmas.append(r)
        for r in amax_rdmas:
            r.wait()

        gmax = jnp.max(amax_ref[...])
        scale = gmax / 448.0
        inv_scale = 448.0 / gmax
        for j in range(N_DEV):
            blk = out_ref[j * m_per:(j + 1) * m_per, :]
            q = (blk * inv_scale).astype(jnp.float8_e4m3fn).astype(jnp.float32)
            out_ref[j * m_per:(j + 1) * m_per, :] = q * scale

    return pl.pallas_call(
        body,
        out_shape=jax.ShapeDtypeStruct((M, n_per), jnp.float32),
        in_specs=[
            pl.BlockSpec(memory_space=pl.ANY),
            pl.BlockSpec(memory_space=pl.ANY),
        ],
        out_specs=pl.BlockSpec(memory_space=pltpu.VMEM),
        scratch_shapes=[
            pltpu.VMEM((M, k), jnp.bfloat16),
            pltpu.VMEM((k, n_per), jnp.bfloat16),
            pltpu.VMEM((2, m_per // XPIECES, k), jnp.float32),
            pltpu.VMEM((2, k // WPIECES, n_per), jnp.float32),
            pltpu.VMEM((N_DEV, 128), jnp.float32),
            pltpu.SemaphoreType.DMA((2,)),
            pltpu.SemaphoreType.DMA((2,)),
            pltpu.SemaphoreType.DMA((12,)),
            pltpu.SemaphoreType.DMA((12,)),
            pltpu.SemaphoreType.DMA((N_DEV - 1,)),
            pltpu.SemaphoreType.DMA((N_DEV - 1,)),
        ],
        compiler_params=pltpu.CompilerParams(
            collective_id=0,
            vmem_limit_bytes=64 * 1024 * 1024,
        ),
    )(x, w_mat)
